# baseline (speedup 1.0000x reference)
import jax
import jax.numpy as jnp
from jax import lax
from jax.experimental import pallas as pl
from jax.experimental.pallas import tpu as pltpu
from jax.experimental.pallas import tpu_sc as plsc

B, P, D = 4, 8192, 768
N = B * P
LANES = 16
NC, NS = 2, 16
NW = NC * NS
RPW = N // NW
C = 16
NCHUNK = RPW // C
GROUPS = D // LANES
PK = D // 32
NBUF = 4
PF = 2
TAIL = NCHUNK % NBUF


def _pe_add_kernel(x_hbm, idx_hbm, pe_hbm, out_hbm, idx_v, *scratch):
    xbufs = scratch[0:NBUF]
    pebufs = scratch[NBUF:2 * NBUF]
    sem_x = scratch[2 * NBUF:3 * NBUF]
    sem_pe = scratch[3 * NBUF:4 * NBUF]
    sem_out = scratch[4 * NBUF:5 * NBUF]

    wid = lax.axis_index("s") * NC + lax.axis_index("c")
    base = wid * RPW
    pltpu.sync_copy(idx_hbm.at[pl.ds(base, RPW)], idx_v)

    def start_in(i, b):
        row0 = base + i * C
        pltpu.make_async_copy(
            x_hbm.at[pl.ds(row0 * D, C * D)], xbufs[b], sem_x[b]).start()
        pltpu.make_async_copy(
            pe_hbm.at[idx_v.at[pl.ds(i * C, C)]], pebufs[b], sem_pe[b]).start()

    def wait_in(i, b):
        row0 = base + i * C
        pltpu.make_async_copy(
            x_hbm.at[pl.ds(row0 * D, C * D)], xbufs[b], sem_x[b]).wait()
        pltpu.make_async_copy(
            pe_hbm.at[idx_v.at[pl.ds(i * C, C)]], pebufs[b], sem_pe[b]).wait()

    def start_out(i, b):
        row0 = base + i * C
        pltpu.make_async_copy(
            xbufs[b], out_hbm.at[pl.ds(row0 * D, C * D)], sem_out[b]).start()

    def wait_out(i, b):
        row0 = base + i * C
        pltpu.make_async_copy(
            xbufs[b], out_hbm.at[pl.ds(row0 * D, C * D)], sem_out[b]).wait()

    def body(i, b, traced):
        nb = (b + PF) % NBUF
        gap = NBUF - PF

        if traced:
            @pl.when(jnp.logical_and(i >= gap, i + PF < NCHUNK))
            def _():
                wait_out(i - gap, nb)

            @pl.when(i + PF < NCHUNK)
            def _():
                start_in(i + PF, nb)
        else:
            if i >= gap and i + PF < NCHUNK:
                wait_out(i - gap, nb)
            if i + PF < NCHUNK:
                start_in(i + PF, nb)

        wait_in(i, b)

        def row_body(r, _):
            rbase = r * D
            for j in range(PK):
                pg = pebufs[b][r, pl.ds(j * LANES, LANES)]
                lo = lax.bitcast_convert_type(pg << 16, jnp.float32)
                hi = lax.bitcast_convert_type(pg & jnp.int32(-65536),
                                              jnp.float32)
                plsc.addupdate(xbufs[b].at[pl.ds(rbase + j * LANES, LANES)],
                               lo)
                plsc.addupdate(
                    xbufs[b].at[pl.ds(rbase + D // 2 + j * LANES, LANES)], hi)
            return 0

        lax.fori_loop(0, C, row_body, 0)
        start_out(i, b)

    for i in range(PF):
        start_in(i, i)

    def outer(i0, _):
        for b in range(NBUF):
            body(i0 + b, b, traced=True)
        return 0

    lax.fori_loop(0, (NCHUNK - TAIL) // NBUF,
                  lambda s, c: outer(s * NBUF, c), 0)

    for t in range(TAIL):
        i = NCHUNK - TAIL + t
        body(i, i % NBUF, traced=False)

    waited = {i - (NBUF - PF) for i in range(NCHUNK)
              if i >= (NBUF - PF) and i + PF < NCHUNK}
    for i in sorted(set(range(NCHUNK)) - waited):
        wait_out(i, i % NBUF)


@jax.jit
def kernel(x, indices, pe):
    x2 = x.reshape(N * D)
    idx = indices.reshape(N)
    pe2 = pe.reshape(P, D)
    lo_u = jax.lax.bitcast_convert_type(
        pe2[:, :D // 2].astype(jnp.bfloat16), jnp.uint16).astype(jnp.uint32)
    hi_u = jax.lax.bitcast_convert_type(
        pe2[:, D // 2:].astype(jnp.bfloat16), jnp.uint16).astype(jnp.uint32)
    tab = jax.lax.bitcast_convert_type(lo_u | (hi_u << 16), jnp.int32)
    mesh = plsc.VectorSubcoreMesh(core_axis_name="c", subcore_axis_name="s")
    out = pl.kernel(
        _pe_add_kernel,
        out_type=jax.ShapeDtypeStruct((N * D,), jnp.float32),
        mesh=mesh,
        scratch_types=(
            [pltpu.VMEM((RPW,), jnp.int32)]
            + [pltpu.VMEM((C * D,), jnp.float32) for _ in range(NBUF)]
            + [pltpu.VMEM((C, D // 2), jnp.int32) for _ in range(NBUF)]
            + [pltpu.SemaphoreType.DMA for _ in range(3 * NBUF)]
        ),
    )(x2, idx, tab)
    return out.reshape(B, P, D)


# speedup vs baseline: 2.7594x; 2.7594x over previous
import jax
import jax.numpy as jnp
from jax import lax
from jax.experimental import pallas as pl
from jax.experimental.pallas import tpu as pltpu
from jax.experimental.pallas import tpu_sc as plsc

B, P, D = 4, 8192, 768
N = B * P
LANES = 16
NC, NS = 2, 16
NW = NC * NS
RPW = N // NW
C = 16
NCHUNK = RPW // C
GROUPS = D // LANES
NBUF = 4


def _pe_add_kernel(x_hbm, idx_hbm, pe_hbm, out_hbm, idx_v, *scratch):
    xbufs = scratch[0:NBUF]
    pebufs = scratch[NBUF:2 * NBUF]
    sem_x = scratch[2 * NBUF:3 * NBUF]
    sem_pe = scratch[3 * NBUF:4 * NBUF]
    sem_out = scratch[4 * NBUF:5 * NBUF]

    wid = lax.axis_index("s") * NC + lax.axis_index("c")
    base = wid * RPW
    pltpu.sync_copy(idx_hbm.at[pl.ds(base, RPW)], idx_v)

    def start_in(i, b):
        row0 = base + i * C
        pltpu.make_async_copy(
            x_hbm.at[pl.ds(row0, C)], xbufs[b], sem_x[b]).start()
        pltpu.make_async_copy(
            pe_hbm.at[idx_v.at[pl.ds(i * C, C)]], pebufs[b], sem_pe[b]).start()

    def wait_in(i, b):
        row0 = base + i * C
        pltpu.make_async_copy(
            x_hbm.at[pl.ds(row0, C)], xbufs[b], sem_x[b]).wait()
        pltpu.make_async_copy(
            pe_hbm.at[idx_v.at[pl.ds(i * C, C)]], pebufs[b], sem_pe[b]).wait()

    def start_out(i, b):
        row0 = base + i * C
        pltpu.make_async_copy(
            xbufs[b], out_hbm.at[pl.ds(row0, C)], sem_out[b]).start()

    def wait_out(i, b):
        row0 = base + i * C
        pltpu.make_async_copy(
            xbufs[b], out_hbm.at[pl.ds(row0, C)], sem_out[b]).wait()

    start_in(0, 0)
    start_in(1, 1)

    def outer(i0, _):
        for b in range(NBUF):
            i = i0 + b
            wait_in(i, b)

            nb = (b + 2) % NBUF

            @pl.when(i >= 2)
            def _():
                wait_out(i - 2, nb)

            @pl.when(i + 2 < NCHUNK)
            def _():
                start_in(i + 2, nb)

            def row_body(r, _):
                for k in range(GROUPS):
                    plsc.addupdate(xbufs[b].at[r, pl.ds(k * LANES, LANES)],
                                   pebufs[b][r, pl.ds(k * LANES, LANES)])
                return 0

            lax.fori_loop(0, C, row_body, 0)
            start_out(i, b)
        return 0

    lax.fori_loop(0, NCHUNK // NBUF, lambda s, c: outer(s * NBUF, c), 0)

    for i in range(NCHUNK - 2, NCHUNK):
        wait_out(i, i % NBUF)


@jax.jit
def kernel(x, indices, pe):
    x2 = x.reshape(N, D)
    idx = indices.reshape(N)
    tab = pe.reshape(P, D)
    mesh = plsc.VectorSubcoreMesh(core_axis_name="c", subcore_axis_name="s")
    out = pl.kernel(
        _pe_add_kernel,
        out_type=jax.ShapeDtypeStruct((N, D), jnp.float32),
        mesh=mesh,
        scratch_types=(
            [pltpu.VMEM((RPW,), jnp.int32)]
            + [pltpu.VMEM((C, D), jnp.float32) for _ in range(NBUF)]
            + [pltpu.VMEM((C, D), jnp.float32) for _ in range(NBUF)]
            + [pltpu.SemaphoreType.DMA for _ in range(3 * NBUF)]
        ),
    )(x2, idx, tab)
    return out.reshape(B, P, D)
